# baseline (speedup 1.0000x reference)
import functools
import jax
import jax.numpy as jnp
from jax import lax
from jax.experimental import pallas as pl
from jax.experimental.pallas import tpu as pltpu
from jax.experimental.pallas import tpu_sc as plsc

NC = 2
NS = 16
NW = NC * NS

C = 16
NB = 3
H = 1024


def _sc_body(x_hbm, pos_hbm, o_hbm, xbuf, pbuf, xsems, psems, osems):
    rows = x_hbm.shape[0]
    seq = pos_hbm.shape[0]
    rpw = rows // NW
    nch = rpw // C
    wid = lax.axis_index("s") * NC + lax.axis_index("c")
    row0 = wid * rpw
    pos0 = lax.rem(row0, seq)

    def in_x(g, slot):
        return pltpu.make_async_copy(
            x_hbm.at[pl.ds(row0 + g * C, C), :], xbuf.at[slot], xsems.at[slot])

    def in_p(g, slot):
        return pltpu.make_async_copy(
            pos_hbm.at[pl.ds(pos0 + g * C, C), :], pbuf.at[slot], psems.at[slot])

    def out_x(g, slot):
        return pltpu.make_async_copy(
            xbuf.at[slot], o_hbm.at[pl.ds(row0 + g * C, C), :], osems.at[slot])

    def compute(slot):
        def row_body(r, carry):
            def vec_body(k, carry2):
                sl = pl.ds(k * 16, 16)
                plsc.addupdate(xbuf.at[slot, r, sl], pbuf[slot, r, sl])
                return carry2
            return lax.fori_loop(0, H // 16, vec_body, carry, unroll=8)
        lax.fori_loop(0, C, row_body, 0)

    for g in range(min(NB - 1, nch)):
        in_x(g, g % NB).start()
        in_p(g, g % NB).start()

    def loop_body(g, carry):
        slot = lax.rem(g, NB)
        look = g + NB - 1
        nslot = lax.rem(look, NB)

        @pl.when(jnp.logical_and(look < nch, g >= 1))
        def _():
            out_x(g - 1, nslot).wait()

        @pl.when(look < nch)
        def _():
            in_x(look, nslot).start()
            in_p(look, nslot).start()

        in_x(g, slot).wait()
        in_p(g, slot).wait()
        compute(slot)
        out_x(g, slot).start()
        return carry

    lax.fori_loop(0, nch, loop_body, 0)

    for g in range(max(0, nch - NB), nch):
        out_x(g, g % NB).wait()


def sc_add(x2d, pos_table):
    rows, hidden = x2d.shape
    mesh = plsc.VectorSubcoreMesh(core_axis_name="c", subcore_axis_name="s", num_cores=NC, num_subcores=NS)
    kern = pl.kernel(
        _sc_body,
        out_type=jax.ShapeDtypeStruct((rows, hidden), jnp.float32),
        mesh=mesh,
        scratch_types=[
            pltpu.VMEM((NB, C, H), jnp.float32),
            pltpu.VMEM((NB, C, H), jnp.float32),
            pltpu.SemaphoreType.DMA((NB,)),
            pltpu.SemaphoreType.DMA((NB,)),
            pltpu.SemaphoreType.DMA((NB,)),
        ],
    )
    return kern(x2d, pos_table)


def kernel(x, pos_table):
    batch, seq_len, hidden = x.shape
    xr = x.reshape(batch * seq_len, hidden)
    out = sc_add(xr, pos_table)
    return out.reshape(batch, seq_len, hidden)
